# baseline (speedup 1.0000x reference)
import functools

import jax
import jax.numpy as jnp
from jax import lax
from jax.experimental import pallas as pl
from jax.experimental.pallas import tpu as pltpu
from jax.experimental.pallas import tpu_sc as plsc


def _build_gather(N, V, D, NC, NS):
    NW = NC * NS
    R = N // NW
    C = 512
    NCHUNK = R // C
    mesh = plsc.VectorSubcoreMesh(core_axis_name="c", subcore_axis_name="s")

    @functools.partial(
        pl.kernel,
        mesh=mesh,
        compiler_params=pltpu.CompilerParams(use_tc_tiling_on_sc=False),
        out_type=jax.ShapeDtypeStruct((N, D), jnp.float32),
        scratch_types=[
            pltpu.VMEM((R,), jnp.int32),
            pltpu.VMEM((C, D), jnp.float32),
            pltpu.SemaphoreType.DMA,
        ],
    )
    def gather_kernel(table_hbm, idx_hbm, out_hbm, idx_v, rows_v, sem):
        wid = lax.axis_index("s") * NC + lax.axis_index("c")
        base = wid * R
        pltpu.sync_copy(idx_hbm.at[pl.ds(base, R)], idx_v)

        def chunk(c, carry):
            off = c * C
            pltpu.async_copy(
                table_hbm.at[idx_v.at[pl.ds(off, C)]], rows_v, sem
            ).wait()
            pltpu.sync_copy(rows_v, out_hbm.at[pl.ds(base + off, C)])
            return carry

        lax.fori_loop(0, NCHUNK, chunk, 0)

    return gather_kernel


def kernel(indices, table):
    B, H = indices.shape
    V, D = table.shape
    N = B * H
    info = plsc.get_sparse_core_info()
    idx_flat = indices.reshape(N).astype(jnp.int32)
    gather = _build_gather(N, V, D, info.num_cores, info.num_subcores)
    out = gather(table, idx_flat)
    return out.reshape(B, H, D)


# speedup vs baseline: 1.0199x; 1.0199x over previous
import functools

import jax
import jax.numpy as jnp
from jax import lax
from jax.experimental import pallas as pl
from jax.experimental.pallas import tpu as pltpu
from jax.experimental.pallas import tpu_sc as plsc

_NBUF = 4
_C = 256


def _build_gather(N, V, D, NC, NS):
    NW = NC * NS
    R = N // NW
    C = _C
    NCHUNK = R // C
    NI = NCHUNK // _NBUF
    mesh = plsc.VectorSubcoreMesh(core_axis_name="c", subcore_axis_name="s")

    @functools.partial(
        pl.kernel,
        mesh=mesh,
        compiler_params=pltpu.CompilerParams(use_tc_tiling_on_sc=False),
        out_type=jax.ShapeDtypeStruct((N, D), jnp.float32),
        scratch_types=[
            pltpu.VMEM((R,), jnp.int32),
            [pltpu.VMEM((C, D), jnp.float32) for _ in range(_NBUF)],
            [pltpu.SemaphoreType.DMA for _ in range(_NBUF)],
            [pltpu.SemaphoreType.DMA for _ in range(_NBUF)],
        ],
    )
    def gather_kernel(table_hbm, idx_hbm, out_hbm, idx_v, bufs, gsems, ssems):
        wid = lax.axis_index("s") * NC + lax.axis_index("c")
        base = wid * R
        pltpu.sync_copy(idx_hbm.at[pl.ds(base, R)], idx_v)

        def start_g(b, off):
            pltpu.async_copy(
                table_hbm.at[idx_v.at[pl.ds(off, C)]], bufs[b], gsems[b]
            )

        def wait_g(b, off):
            pltpu.make_async_copy(
                table_hbm.at[idx_v.at[pl.ds(off, C)]], bufs[b], gsems[b]
            ).wait()

        def start_s(b, off):
            pltpu.async_copy(bufs[b], out_hbm.at[pl.ds(base + off, C)], ssems[b])

        def wait_s(b, off):
            pltpu.make_async_copy(
                bufs[b], out_hbm.at[pl.ds(base + off, C)], ssems[b]
            ).wait()

        start_g(0, 0)
        start_g(1, C)

        def body(i, carry):
            c0 = i * (_NBUF * C)
            for b in range(_NBUF):
                off = c0 + b * C
                pb = (b + 2) % _NBUF
                poff = off + 2 * C
                if b < 2:
                    @pl.when(i > 0)
                    def _():
                        wait_s(pb, poff - _NBUF * C)
                        start_g(pb, poff)

                    @pl.when(i == 0)
                    def _():
                        start_g(pb, poff)
                else:
                    wait_s(pb, poff - _NBUF * C)

                    @pl.when(i < NI - 1)
                    def _():
                        start_g(pb, poff)
                wait_g(b, off)
                start_s(b, off)
            return carry

        lax.fori_loop(0, NI, body, 0)
        wait_s(2, (NCHUNK - 2) * C)
        wait_s(3, (NCHUNK - 1) * C)

    return gather_kernel


def kernel(indices, table):
    B, H = indices.shape
    V, D = table.shape
    N = B * H
    info = plsc.get_sparse_core_info()
    idx_flat = indices.reshape(N).astype(jnp.int32)
    gather = _build_gather(N, V, D, info.num_cores, info.num_subcores)
    out = gather(table, idx_flat)
    return out.reshape(B, H, D)


# speedup vs baseline: 1.0236x; 1.0036x over previous
import functools

import jax
import jax.numpy as jnp
from jax import lax
from jax.experimental import pallas as pl
from jax.experimental.pallas import tpu as pltpu
from jax.experimental.pallas import tpu_sc as plsc

_NBUF = 4


def _build_gather(B, H, V, D, NC, NS):
    NW = NC * NS
    RB = B // NW
    NI = RB // _NBUF
    mesh = plsc.VectorSubcoreMesh(core_axis_name="c", subcore_axis_name="s")

    @functools.partial(
        pl.kernel,
        mesh=mesh,
        compiler_params=pltpu.CompilerParams(use_tc_tiling_on_sc=False),
        out_type=jax.ShapeDtypeStruct((B, H, D), jnp.float32),
        scratch_types=[
            pltpu.VMEM((RB, H), jnp.int32),
            [pltpu.VMEM((H, D), jnp.float32) for _ in range(_NBUF)],
            [pltpu.SemaphoreType.DMA for _ in range(_NBUF)],
            [pltpu.SemaphoreType.DMA for _ in range(_NBUF)],
        ],
    )
    def gather_kernel(table_hbm, idx_hbm, out_hbm, idx_v, bufs, gsems, ssems):
        wid = lax.axis_index("s") * NC + lax.axis_index("c")
        base = wid * RB
        pltpu.sync_copy(idx_hbm.at[pl.ds(base, RB)], idx_v)

        def start_g(b, r):
            pltpu.async_copy(table_hbm.at[idx_v.at[r]], bufs[b], gsems[b])

        def wait_g(b, r):
            pltpu.make_async_copy(
                table_hbm.at[idx_v.at[r]], bufs[b], gsems[b]
            ).wait()

        def start_s(b, r):
            pltpu.async_copy(bufs[b], out_hbm.at[base + r], ssems[b])

        def wait_s(b, r):
            pltpu.make_async_copy(bufs[b], out_hbm.at[base + r], ssems[b]).wait()

        start_g(0, 0)
        start_g(1, 1)

        def body(i, carry):
            r0 = i * _NBUF
            for b in range(_NBUF):
                r = r0 + b
                pb = (b + 2) % _NBUF
                pr = r + 2
                if b < 2:
                    @pl.when(i > 0)
                    def _():
                        wait_s(pb, pr - _NBUF)
                        start_g(pb, pr)

                    @pl.when(i == 0)
                    def _():
                        start_g(pb, pr)
                else:
                    wait_s(pb, pr - _NBUF)

                    @pl.when(i < NI - 1)
                    def _():
                        start_g(pb, pr)
                wait_g(b, r)
                start_s(b, r)
            return carry

        lax.fori_loop(0, NI, body, 0)
        wait_s(2, RB - 2)
        wait_s(3, RB - 1)

    return gather_kernel


def kernel(indices, table):
    B, H = indices.shape
    V, D = table.shape
    info = plsc.get_sparse_core_info()
    gather = _build_gather(B, H, V, D, info.num_cores, info.num_subcores)
    return gather(table, indices.astype(jnp.int32))


# speedup vs baseline: 1.3594x; 1.3281x over previous
import functools

import jax
import jax.numpy as jnp
from jax import lax
from jax.experimental import pallas as pl
from jax.experimental.pallas import tpu as pltpu
from jax.experimental.pallas import tpu_sc as plsc

_NBUF = 4


def _build_gather(B, H, V, D, NC, NS):
    NW = NC * NS
    RB = B // NW
    NI = RB // _NBUF
    mesh = plsc.VectorSubcoreMesh(core_axis_name="c", subcore_axis_name="s")

    @functools.partial(
        pl.kernel,
        mesh=mesh,
        compiler_params=pltpu.CompilerParams(use_tc_tiling_on_sc=False),
        out_type=jax.ShapeDtypeStruct((B * H, 2 * D), jnp.float32),
        scratch_types=[
            pltpu.VMEM((RB, H), jnp.int32),
            [pltpu.VMEM((H, D), jnp.float32) for _ in range(_NBUF)],
            [pltpu.SemaphoreType.DMA for _ in range(_NBUF)],
            [pltpu.SemaphoreType.DMA for _ in range(_NBUF)],
        ],
    )
    def gather_kernel(table_hbm, idx_hbm, out_hbm, idx_v, bufs, gsems, ssems):
        wid = lax.axis_index("s") * NC + lax.axis_index("c")
        base = wid * RB
        pltpu.sync_copy(idx_hbm.at[pl.ds(base, RB)], idx_v)

        def start_g(b, r):
            pltpu.async_copy(table_hbm.at[idx_v.at[r]], bufs[b], gsems[b])

        def wait_g(b, r):
            pltpu.make_async_copy(
                table_hbm.at[idx_v.at[r]], bufs[b], gsems[b]
            ).wait()

        def start_s(b, r):
            pltpu.async_copy(
                bufs[b],
                out_hbm.at[pl.ds((base + r) * H, H), pl.ds(0, D)],
                ssems[b],
            )

        def wait_s(b, r):
            pltpu.make_async_copy(
                bufs[b],
                out_hbm.at[pl.ds((base + r) * H, H), pl.ds(0, D)],
                ssems[b],
            ).wait()

        start_g(0, 0)
        start_g(1, 1)

        def body(i, carry):
            r0 = i * _NBUF
            for b in range(_NBUF):
                r = r0 + b
                pb = (b + 2) % _NBUF
                pr = r + 2
                if b < 2:
                    @pl.when(i > 0)
                    def _():
                        wait_s(pb, pr - _NBUF)
                        start_g(pb, pr)

                    @pl.when(i == 0)
                    def _():
                        start_g(pb, pr)
                else:
                    wait_s(pb, pr - _NBUF)

                    @pl.when(i < NI - 1)
                    def _():
                        start_g(pb, pr)
                wait_g(b, r)
                start_s(b, r)
            return carry

        lax.fori_loop(0, NI, body, 0)
        wait_s(2, RB - 2)
        wait_s(3, RB - 1)

    return gather_kernel


def kernel(indices, table):
    B, H = indices.shape
    V, D = table.shape
    info = plsc.get_sparse_core_info()
    gather = _build_gather(B, H, V, D, info.num_cores, info.num_subcores)
    padded = gather(table, indices.astype(jnp.int32))
    return padded[:, :D].reshape(B, H, D)
